# baseline (speedup 1.0000x reference)
import functools
import jax
import jax.numpy as jnp
from jax import lax
from jax.experimental import pallas as pl
from jax.experimental.pallas import tpu as pltpu
from jax.experimental.pallas import tpu_sc as plsc

NC, NS = 2, 16
NW = NC * NS
CHUNK = 128
LANES = 16


def _build(V, D, N, S):
    n_per_w = N // NW
    n_chunks = n_per_w // CHUNK
    mesh = plsc.VectorSubcoreMesh(core_axis_name="c", subcore_axis_name="s")

    @functools.partial(
        pl.kernel,
        out_type=jax.ShapeDtypeStruct((N, D), jnp.float32),
        mesh=mesh,
        compiler_params=pltpu.CompilerParams(use_tc_tiling_on_sc=False),
        scratch_types=[
            pltpu.VMEM((n_per_w,), jnp.int32),
            pltpu.VMEM((S * D,), jnp.float32),
            pltpu.VMEM((CHUNK, D), jnp.float32),
            pltpu.SemaphoreType.DMA,
        ],
    )
    def emb(idx_hbm, table_hbm, pos_hbm, out_hbm, idx_v, pos_v, rows_v, gsem):
        wid = lax.axis_index("s") * NC + lax.axis_index("c")
        base = wid * n_per_w
        pltpu.sync_copy(idx_hbm.at[pl.ds(base, n_per_w)], idx_v)
        pltpu.sync_copy(pos_hbm, pos_v)

        def chunk_body(ci, _):
            g0 = base + ci * CHUNK
            pltpu.async_copy(
                table_hbm.at[idx_v.at[pl.ds(ci * CHUNK, CHUNK)]], rows_v, gsem
            ).wait()
            p0 = lax.rem(g0, S)

            def row_body(j, p):
                for k in range(D // LANES):
                    plsc.addupdate(
                        rows_v.at[j, pl.ds(k * LANES, LANES)],
                        pos_v[pl.ds(p * D + k * LANES, LANES)],
                    )
                return jnp.where(p + 1 >= S, 0, p + 1)

            lax.fori_loop(0, CHUNK, row_body, p0)
            pltpu.sync_copy(rows_v, out_hbm.at[pl.ds(g0, CHUNK)])
            return _

        lax.fori_loop(0, n_chunks, chunk_body, 0)

    return emb


def kernel(x, token_table, pos_embedding):
    B, S = x.shape
    V, D = token_table.shape
    N = B * S
    idx = x.reshape(N).astype(jnp.int32)
    pos = pos_embedding.reshape(S * D).astype(jnp.float32)
    emb = _build(V, D, N, S)
    out = emb(idx, token_table, pos)
    return out.reshape(B, S, D)


# speedup vs baseline: 1.0585x; 1.0585x over previous
import functools
import jax
import jax.numpy as jnp
from jax import lax
from jax.experimental import pallas as pl
from jax.experimental.pallas import tpu as pltpu
from jax.experimental.pallas import tpu_sc as plsc

NC, NS = 2, 16
NW = NC * NS
CHUNK = 128
LANES = 16
NBUF = 5


def _build(V, D, N, S):
    n_per_w = N // NW
    n_chunks = n_per_w // CHUNK
    n_groups = n_chunks // NBUF
    mesh = plsc.VectorSubcoreMesh(core_axis_name="c", subcore_axis_name="s")

    @functools.partial(
        pl.kernel,
        out_type=jax.ShapeDtypeStruct((N, D), jnp.float32),
        mesh=mesh,
        compiler_params=pltpu.CompilerParams(use_tc_tiling_on_sc=False),
        scratch_types=[
            pltpu.VMEM((n_per_w,), jnp.int32),
            pltpu.VMEM((S * D,), jnp.float32),
            pltpu.VMEM((NBUF, CHUNK, D), jnp.float32),
            pltpu.SemaphoreType.DMA((NBUF,)),
            pltpu.SemaphoreType.DMA((NBUF,)),
        ],
    )
    def emb(idx_hbm, table_hbm, pos_hbm, out_hbm, idx_v, pos_v, rows_v,
            gsem, osem):
        wid = lax.axis_index("s") * NC + lax.axis_index("c")
        base = wid * n_per_w
        pltpu.sync_copy(idx_hbm.at[pl.ds(base, n_per_w)], idx_v)
        pltpu.sync_copy(pos_hbm, pos_v)

        def gather(ci, b):
            return pltpu.async_copy(
                table_hbm.at[idx_v.at[pl.ds(ci * CHUNK, CHUNK)]],
                rows_v.at[b], gsem.at[b])

        def add_pos(ci, b):
            p0 = lax.rem((base + ci * CHUNK), S)

            def row_body(j, p):
                for k in range(D // LANES):
                    plsc.addupdate(
                        rows_v.at[b, j, pl.ds(k * LANES, LANES)],
                        pos_v[pl.ds(p * D + k * LANES, LANES)],
                    )
                return jnp.where(p + 1 >= S, 0, p + 1)

            lax.fori_loop(0, CHUNK, row_body, p0)

        def group_body(cg, _):
            c0 = cg * NBUF
            handles = []
            for b in range(NBUF):
                @pl.when(cg > 0)
                def _drain(b=b):
                    pltpu.make_async_copy(
                        rows_v.at[b], out_hbm.at[pl.ds(base, CHUNK)],
                        osem.at[b]).wait()
                handles.append(gather(c0 + b, b))
            for b in range(NBUF):
                handles[b].wait()
                add_pos(c0 + b, b)
                pltpu.async_copy(
                    rows_v.at[b],
                    out_hbm.at[pl.ds(base + (c0 + b) * CHUNK, CHUNK)],
                    osem.at[b])
            return _

        lax.fori_loop(0, n_groups, group_body, 0)
        for b in range(NBUF):
            pltpu.make_async_copy(
                rows_v.at[b], out_hbm.at[pl.ds(base, CHUNK)],
                osem.at[b]).wait()

    return emb


def kernel(x, token_table, pos_embedding):
    B, S = x.shape
    V, D = token_table.shape
    N = B * S
    idx = x.reshape(N).astype(jnp.int32)
    pos = pos_embedding.reshape(S * D).astype(jnp.float32)
    emb = _build(V, D, N, S)
    out = emb(idx, token_table, pos)
    return out.reshape(B, S, D)
